# Initial kernel scaffold; baseline (speedup 1.0000x reference)
import jax
import jax.numpy as jnp
from jax.experimental import pallas as pl


def kernel(in_feat, edge_index, W, b):
    raise NotImplementedError("write your pallas kernel here")


# speedup vs baseline: 21.9227x; 21.9227x over previous
import functools

import jax
import jax.numpy as jnp
from jax import lax
from jax.experimental import pallas as pl
from jax.experimental.pallas import tpu as pltpu
from jax.experimental.pallas import tpu_sc as plsc

N_NODES = 10000
N_EDGES = 320000
IN_FEATS = 128
NUM_CLASSES = 16

N_PAD = 10240
NW = 32
STEPS = 80
LANES = 128
E_PER_TILE = STEPS * LANES
E_PAD = NW * E_PER_TILE
ROWS_PER_TILE = N_PAD // 16

_mesh = plsc.VectorSubcoreMesh(core_axis_name="c", subcore_axis_name="s")


@functools.partial(
    pl.kernel,
    mesh=_mesh,
    out_type=[
        jax.ShapeDtypeStruct((NW, ROWS_PER_TILE, NUM_CLASSES), jnp.float32),
        jax.ShapeDtypeStruct((NW, ROWS_PER_TILE, NUM_CLASSES), jnp.float32),
    ],
    scratch_types=[
        pltpu.VMEM((STEPS, LANES), jnp.int32),
        pltpu.VMEM((STEPS, LANES), jnp.int32),
        pltpu.VMEM((LANES, NUM_CLASSES), jnp.float32),
        pltpu.VMEM((ROWS_PER_TILE, NUM_CLASSES), jnp.float32),
        pltpu.VMEM_SHARED((N_PAD, NUM_CLASSES), jnp.float32),
        pltpu.VMEM_SHARED((N_PAD, NUM_CLASSES), jnp.float32),
    ],
    compiler_params=pltpu.CompilerParams(use_tc_tiling_on_sc=False),
)
def _deg(src_hbm, dst_hbm, out_s, out_d, idx_s, idx_d, ones_v, zeros_v,
         hist_s, hist_d):
    cid = lax.axis_index("c")
    sid = lax.axis_index("s")
    gid = cid * 16 + sid

    def fill(i, _):
        ones_v[i] = jnp.ones((16,), jnp.float32)
        return 0
    lax.fori_loop(0, LANES, fill, 0)

    def fillz(i, _):
        zeros_v[i] = jnp.zeros((16,), jnp.float32)
        return 0
    lax.fori_loop(0, ROWS_PER_TILE, fillz, 0)

    base = pl.multiple_of(sid * ROWS_PER_TILE, 128)
    pltpu.sync_copy(zeros_v, hist_s.at[pl.ds(base, ROWS_PER_TILE)])
    pltpu.sync_copy(zeros_v, hist_d.at[pl.ds(base, ROWS_PER_TILE)])
    pltpu.sync_copy(src_hbm.at[gid], idx_s)
    pltpu.sync_copy(dst_hbm.at[gid], idx_d)
    plsc.subcore_barrier()

    def body(s, _):
        pltpu.sync_copy(ones_v, hist_s.at[idx_s.at[s]], add=True)
        pltpu.sync_copy(ones_v, hist_d.at[idx_d.at[s]], add=True)
        return 0
    lax.fori_loop(0, STEPS, body, 0)

    plsc.subcore_barrier()
    pltpu.sync_copy(hist_s.at[pl.ds(base, ROWS_PER_TILE)], out_s.at[gid])
    pltpu.sync_copy(hist_d.at[pl.ds(base, ROWS_PER_TILE)], out_d.at[gid])


@functools.partial(
    pl.kernel,
    mesh=_mesh,
    out_type=jax.ShapeDtypeStruct((NW, ROWS_PER_TILE, NUM_CLASSES),
                                  jnp.float32),
    scratch_types=[
        pltpu.VMEM((STEPS, LANES), jnp.int32),
        pltpu.VMEM((STEPS, LANES), jnp.int32),
        pltpu.VMEM((LANES, NUM_CLASSES), jnp.float32),
        pltpu.VMEM((ROWS_PER_TILE, NUM_CLASSES), jnp.float32),
        pltpu.VMEM_SHARED((N_PAD, NUM_CLASSES), jnp.float32),
        pltpu.SemaphoreType.DMA,
    ],
    compiler_params=pltpu.CompilerParams(use_tc_tiling_on_sc=False),
)
def _agg(y_hbm, src_hbm, dst_hbm, out_hbm, idx_s, idx_d, rows, zeros_v,
         agg, sem):
    cid = lax.axis_index("c")
    sid = lax.axis_index("s")
    gid = cid * 16 + sid

    def fill(i, _):
        zeros_v[i] = jnp.zeros((16,), jnp.float32)
        return 0
    lax.fori_loop(0, ROWS_PER_TILE, fill, 0)

    base = pl.multiple_of(sid * ROWS_PER_TILE, 128)
    pltpu.sync_copy(zeros_v, agg.at[pl.ds(base, ROWS_PER_TILE)])
    pltpu.sync_copy(src_hbm.at[gid], idx_s)
    pltpu.sync_copy(dst_hbm.at[gid], idx_d)
    plsc.subcore_barrier()

    def body(s, _):
        pltpu.async_copy(y_hbm.at[idx_s.at[s]], rows, sem).wait()
        pltpu.sync_copy(rows, agg.at[idx_d.at[s]], add=True)
        return 0
    lax.fori_loop(0, STEPS, body, 0)

    plsc.subcore_barrier()
    pltpu.sync_copy(agg.at[pl.ds(base, ROWS_PER_TILE)], out_hbm.at[gid])


def _mm_body(x_ref, w_ref, dsrc_ref, y_ref):
    d = dsrc_ref[0, :, :1] + dsrc_ref[1, :, :1]
    norm = jnp.where(d > 0, lax.rsqrt(jnp.maximum(d, 1.0)), 0.0)
    y_ref[...] = jnp.dot(x_ref[...], w_ref[...],
                         preferred_element_type=jnp.float32) * norm


def _comb_body(agg_ref, ddst_ref, b_ref, h_ref):
    a = agg_ref[0] + agg_ref[1]
    d = ddst_ref[0, :, :1] + ddst_ref[1, :, :1]
    norm = jnp.where(d > 0, lax.rsqrt(jnp.maximum(d, 1.0)), 0.0)
    h_ref[...] = jnp.maximum(a * norm + b_ref[...], 0.0)


_BLK = 1024
_GRID = N_PAD // _BLK


def kernel(in_feat, edge_index, W, b):
    src = edge_index[0]
    dst = edge_index[1]
    pad_e = E_PAD - N_EDGES
    pad_idx = jnp.full((pad_e,), N_PAD - 1, jnp.int32)
    src3 = jnp.concatenate([src, pad_idx]).reshape(NW, STEPS, LANES)
    dst3 = jnp.concatenate([dst, pad_idx]).reshape(NW, STEPS, LANES)
    x_pad = jnp.pad(in_feat, ((0, N_PAD - N_NODES), (0, 0)))

    deg_s_r, deg_d_r = _deg(src3, dst3)
    dsrc = deg_s_r.reshape(2, N_PAD, NUM_CLASSES)
    ddst = deg_d_r.reshape(2, N_PAD, NUM_CLASSES)

    y = pl.pallas_call(
        _mm_body,
        grid=(_GRID,),
        in_specs=[
            pl.BlockSpec((_BLK, IN_FEATS), lambda i: (i, 0)),
            pl.BlockSpec((IN_FEATS, NUM_CLASSES), lambda i: (0, 0)),
            pl.BlockSpec((2, _BLK, NUM_CLASSES), lambda i: (0, i, 0)),
        ],
        out_specs=pl.BlockSpec((_BLK, NUM_CLASSES), lambda i: (i, 0)),
        out_shape=jax.ShapeDtypeStruct((N_PAD, NUM_CLASSES), jnp.float32),
    )(x_pad, W, dsrc)

    agg_r = _agg(y, src3, dst3)
    agg3 = agg_r.reshape(2, N_PAD, NUM_CLASSES)

    h = pl.pallas_call(
        _comb_body,
        grid=(_GRID,),
        in_specs=[
            pl.BlockSpec((2, _BLK, NUM_CLASSES), lambda i: (0, i, 0)),
            pl.BlockSpec((2, _BLK, NUM_CLASSES), lambda i: (0, i, 0)),
            pl.BlockSpec((1, NUM_CLASSES), lambda i: (0, 0)),
        ],
        out_specs=pl.BlockSpec((_BLK, NUM_CLASSES), lambda i: (i, 0)),
        out_shape=jax.ShapeDtypeStruct((N_PAD, NUM_CLASSES), jnp.float32),
    )(agg3, ddst, b.reshape(1, NUM_CLASSES))

    return h[:N_NODES]


# speedup vs baseline: 26.4872x; 1.2082x over previous
import functools

import jax
import jax.numpy as jnp
from jax import lax
from jax.experimental import pallas as pl
from jax.experimental.pallas import tpu as pltpu
from jax.experimental.pallas import tpu_sc as plsc

N_NODES = 10000
N_EDGES = 320000
IN_FEATS = 128
NUM_CLASSES = 16

N_PAD = 10240
NW = 32
STEPS = 80
LANES = 128
E_PER_TILE = STEPS * LANES
E_PAD = NW * E_PER_TILE
ROWS_PER_TILE = N_PAD // 16

_mesh = plsc.VectorSubcoreMesh(core_axis_name="c", subcore_axis_name="s")


@functools.partial(
    pl.kernel,
    mesh=_mesh,
    out_type=[
        jax.ShapeDtypeStruct((NW, ROWS_PER_TILE, NUM_CLASSES), jnp.float32),
        jax.ShapeDtypeStruct((NW, ROWS_PER_TILE, NUM_CLASSES), jnp.float32),
    ],
    scratch_types=[
        pltpu.VMEM((STEPS, LANES), jnp.int32),
        pltpu.VMEM((STEPS, LANES), jnp.int32),
        pltpu.VMEM((LANES, NUM_CLASSES), jnp.float32),
        pltpu.VMEM((ROWS_PER_TILE, NUM_CLASSES), jnp.float32),
        pltpu.VMEM_SHARED((N_PAD, NUM_CLASSES), jnp.float32),
        pltpu.VMEM_SHARED((N_PAD, NUM_CLASSES), jnp.float32),
        pltpu.SemaphoreType.DMA,
    ],
    compiler_params=pltpu.CompilerParams(use_tc_tiling_on_sc=False),
)
def _deg(src_hbm, dst_hbm, out_s, out_d, idx_s, idx_d, ones_v, zeros_v,
         hist_s, hist_d, dsem):
    cid = lax.axis_index("c")
    sid = lax.axis_index("s")
    gid = cid * 16 + sid

    def fill(i, _):
        ones_v[i] = jnp.ones((16,), jnp.float32)
        return 0
    lax.fori_loop(0, LANES, fill, 0)

    def fillz(i, _):
        zeros_v[i] = jnp.zeros((16,), jnp.float32)
        return 0
    lax.fori_loop(0, ROWS_PER_TILE, fillz, 0)

    base = pl.multiple_of(sid * ROWS_PER_TILE, 128)
    pltpu.sync_copy(zeros_v, hist_s.at[pl.ds(base, ROWS_PER_TILE)])
    pltpu.sync_copy(zeros_v, hist_d.at[pl.ds(base, ROWS_PER_TILE)])
    pltpu.sync_copy(src_hbm.at[gid], idx_s)
    pltpu.sync_copy(dst_hbm.at[gid], idx_d)
    plsc.subcore_barrier()

    def body(s, _):
        d1 = pltpu.async_copy(ones_v, hist_s.at[idx_s.at[s]], dsem, add=True)
        pltpu.sync_copy(ones_v, hist_d.at[idx_d.at[s]], add=True)
        d1.wait()
        return 0
    lax.fori_loop(0, STEPS, body, 0)

    plsc.subcore_barrier()
    pltpu.sync_copy(hist_s.at[pl.ds(base, ROWS_PER_TILE)], out_s.at[gid])
    pltpu.sync_copy(hist_d.at[pl.ds(base, ROWS_PER_TILE)], out_d.at[gid])


@functools.partial(
    pl.kernel,
    mesh=_mesh,
    out_type=jax.ShapeDtypeStruct((NW, ROWS_PER_TILE, NUM_CLASSES),
                                  jnp.float32),
    scratch_types=[
        pltpu.VMEM((STEPS, LANES), jnp.int32),
        pltpu.VMEM((STEPS, LANES), jnp.int32),
        pltpu.VMEM((4, LANES, NUM_CLASSES), jnp.float32),
        pltpu.VMEM((ROWS_PER_TILE, NUM_CLASSES), jnp.float32),
        pltpu.VMEM_SHARED((N_PAD, NUM_CLASSES), jnp.float32),
        pltpu.SemaphoreType.DMA,
    ],
    compiler_params=pltpu.CompilerParams(use_tc_tiling_on_sc=False),
)
def _agg(y_hbm, src_hbm, dst_hbm, out_hbm, idx_s, idx_d, rows, zeros_v,
         agg, sem):
    cid = lax.axis_index("c")
    sid = lax.axis_index("s")
    gid = cid * 16 + sid
    nbuf = 4

    def fill(i, _):
        zeros_v[i] = jnp.zeros((16,), jnp.float32)
        return 0
    lax.fori_loop(0, ROWS_PER_TILE, fill, 0)

    base = pl.multiple_of(sid * ROWS_PER_TILE, 128)
    pltpu.sync_copy(zeros_v, agg.at[pl.ds(base, ROWS_PER_TILE)])
    pltpu.sync_copy(src_hbm.at[gid], idx_s)
    pltpu.sync_copy(dst_hbm.at[gid], idx_d)
    plsc.subcore_barrier()

    for d in range(nbuf):
        pltpu.async_copy(y_hbm.at[idx_s.at[d]], rows.at[d], sem)

    def body(s, _):
        buf = lax.rem(s, nbuf)
        pltpu.make_async_copy(y_hbm.at[idx_s.at[s]], rows.at[buf], sem).wait()
        pltpu.sync_copy(rows.at[buf], agg.at[idx_d.at[s]], add=True)

        @pl.when(s + nbuf < STEPS)
        def _():
            pltpu.async_copy(y_hbm.at[idx_s.at[s + nbuf]], rows.at[buf], sem)
        return 0
    lax.fori_loop(0, STEPS, body, 0)

    plsc.subcore_barrier()
    pltpu.sync_copy(agg.at[pl.ds(base, ROWS_PER_TILE)], out_hbm.at[gid])


def _mm_body(x_ref, w_ref, dsrc_ref, y_ref):
    d = dsrc_ref[0, :, :1] + dsrc_ref[1, :, :1]
    norm = jnp.where(d > 0, lax.rsqrt(jnp.maximum(d, 1.0)), 0.0)
    y_ref[...] = jnp.dot(x_ref[...], w_ref[...],
                         preferred_element_type=jnp.float32) * norm


def _comb_body(agg_ref, ddst_ref, b_ref, h_ref):
    a = agg_ref[0] + agg_ref[1]
    d = ddst_ref[0, :, :1] + ddst_ref[1, :, :1]
    norm = jnp.where(d > 0, lax.rsqrt(jnp.maximum(d, 1.0)), 0.0)
    h_ref[...] = jnp.maximum(a * norm + b_ref[...], 0.0)


_BLK = 1024
_GRID = N_PAD // _BLK


def kernel(in_feat, edge_index, W, b):
    src = edge_index[0]
    dst = edge_index[1]
    pad_e = E_PAD - N_EDGES
    pad_idx = jnp.full((pad_e,), N_PAD - 1, jnp.int32)
    src3 = jnp.concatenate([src, pad_idx]).reshape(NW, STEPS, LANES)
    dst3 = jnp.concatenate([dst, pad_idx]).reshape(NW, STEPS, LANES)
    x_pad = jnp.pad(in_feat, ((0, N_PAD - N_NODES), (0, 0)))

    deg_s_r, deg_d_r = _deg(src3, dst3)
    dsrc = deg_s_r.reshape(2, N_PAD, NUM_CLASSES)
    ddst = deg_d_r.reshape(2, N_PAD, NUM_CLASSES)

    y = pl.pallas_call(
        _mm_body,
        grid=(_GRID,),
        in_specs=[
            pl.BlockSpec((_BLK, IN_FEATS), lambda i: (i, 0)),
            pl.BlockSpec((IN_FEATS, NUM_CLASSES), lambda i: (0, 0)),
            pl.BlockSpec((2, _BLK, NUM_CLASSES), lambda i: (0, i, 0)),
        ],
        out_specs=pl.BlockSpec((_BLK, NUM_CLASSES), lambda i: (i, 0)),
        out_shape=jax.ShapeDtypeStruct((N_PAD, NUM_CLASSES), jnp.float32),
    )(x_pad, W, dsrc)

    agg_r = _agg(y, src3, dst3)
    agg3 = agg_r.reshape(2, N_PAD, NUM_CLASSES)

    h = pl.pallas_call(
        _comb_body,
        grid=(_GRID,),
        in_specs=[
            pl.BlockSpec((2, _BLK, NUM_CLASSES), lambda i: (0, i, 0)),
            pl.BlockSpec((2, _BLK, NUM_CLASSES), lambda i: (0, i, 0)),
            pl.BlockSpec((1, NUM_CLASSES), lambda i: (0, 0)),
        ],
        out_specs=pl.BlockSpec((_BLK, NUM_CLASSES), lambda i: (i, 0)),
        out_shape=jax.ShapeDtypeStruct((N_PAD, NUM_CLASSES), jnp.float32),
    )(agg3, ddst, b.reshape(1, NUM_CLASSES))

    return h[:N_NODES]


# speedup vs baseline: 26.8219x; 1.0126x over previous
import functools

import jax
import jax.numpy as jnp
from jax import lax
from jax.experimental import pallas as pl
from jax.experimental.pallas import tpu as pltpu
from jax.experimental.pallas import tpu_sc as plsc

N_NODES = 10000
N_EDGES = 320000
IN_FEATS = 128
NUM_CLASSES = 16

N_PAD = 10240
NW = 32
STEPS = 80
LANES = 128
E_PER_TILE = STEPS * LANES
E_PAD = NW * E_PER_TILE
ROWS_PER_TILE = N_PAD // 16
NBUF = 8

_mesh = plsc.VectorSubcoreMesh(core_axis_name="c", subcore_axis_name="s")
_sc_params = pltpu.CompilerParams(use_tc_tiling_on_sc=False)


@functools.partial(
    pl.kernel,
    mesh=_mesh,
    out_type=jax.ShapeDtypeStruct((NW, ROWS_PER_TILE, NUM_CLASSES),
                                  jnp.float32),
    scratch_types=[
        pltpu.VMEM((STEPS, LANES), jnp.int32),
        pltpu.VMEM((LANES, NUM_CLASSES), jnp.float32),
        pltpu.VMEM((ROWS_PER_TILE, NUM_CLASSES), jnp.float32),
        pltpu.VMEM_SHARED((N_PAD, NUM_CLASSES), jnp.float32),
    ],
    compiler_params=_sc_params,
)
def _deg(src_hbm, out_s, idx_s, ones_v, zeros_v, hist_s):
    cid = lax.axis_index("c")
    sid = lax.axis_index("s")
    gid = cid * 16 + sid

    def fill(i, _):
        ones_v[i] = jnp.ones((16,), jnp.float32)
        zeros_v[i] = jnp.zeros((16,), jnp.float32)
        return 0
    lax.fori_loop(0, LANES, fill, 0)

    def fillz(i, _):
        zeros_v[i] = jnp.zeros((16,), jnp.float32)
        return 0
    lax.fori_loop(LANES, ROWS_PER_TILE, fillz, 0)

    base = pl.multiple_of(sid * ROWS_PER_TILE, 128)
    pltpu.sync_copy(zeros_v, hist_s.at[pl.ds(base, ROWS_PER_TILE)])
    pltpu.sync_copy(src_hbm.at[gid], idx_s)
    plsc.subcore_barrier()

    def body(s, _):
        pltpu.sync_copy(ones_v, hist_s.at[idx_s.at[s]], add=True)
        return 0
    lax.fori_loop(0, STEPS, body, 0)

    plsc.subcore_barrier()
    pltpu.sync_copy(hist_s.at[pl.ds(base, ROWS_PER_TILE)], out_s.at[gid])


@functools.partial(
    pl.kernel,
    mesh=_mesh,
    out_type=[
        jax.ShapeDtypeStruct((NW, ROWS_PER_TILE, NUM_CLASSES), jnp.float32),
        jax.ShapeDtypeStruct((NW, ROWS_PER_TILE, NUM_CLASSES), jnp.float32),
    ],
    scratch_types=[
        pltpu.VMEM((STEPS, LANES), jnp.int32),
        pltpu.VMEM((STEPS, LANES), jnp.int32),
        pltpu.VMEM((NBUF, LANES, NUM_CLASSES), jnp.float32),
        pltpu.VMEM((LANES, NUM_CLASSES), jnp.float32),
        pltpu.VMEM((ROWS_PER_TILE, NUM_CLASSES), jnp.float32),
        pltpu.VMEM_SHARED((N_PAD, NUM_CLASSES), jnp.float32),
        pltpu.VMEM_SHARED((N_PAD, NUM_CLASSES), jnp.float32),
        pltpu.SemaphoreType.DMA,
        pltpu.SemaphoreType.DMA,
    ],
    compiler_params=_sc_params,
)
def _agg(y_hbm, src_hbm, dst_hbm, out_hbm, out_d, idx_s, idx_d, rows,
         ones_v, zeros_v, agg, hist_d, sem, dsem):
    cid = lax.axis_index("c")
    sid = lax.axis_index("s")
    gid = cid * 16 + sid

    def fill(i, _):
        ones_v[i] = jnp.ones((16,), jnp.float32)
        zeros_v[i] = jnp.zeros((16,), jnp.float32)
        return 0
    lax.fori_loop(0, LANES, fill, 0)

    def fillz(i, _):
        zeros_v[i] = jnp.zeros((16,), jnp.float32)
        return 0
    lax.fori_loop(LANES, ROWS_PER_TILE, fillz, 0)

    base = pl.multiple_of(sid * ROWS_PER_TILE, 128)
    pltpu.sync_copy(zeros_v, agg.at[pl.ds(base, ROWS_PER_TILE)])
    pltpu.sync_copy(zeros_v, hist_d.at[pl.ds(base, ROWS_PER_TILE)])
    pltpu.sync_copy(src_hbm.at[gid], idx_s)
    pltpu.sync_copy(dst_hbm.at[gid], idx_d)
    plsc.subcore_barrier()

    for d in range(NBUF):
        pltpu.async_copy(y_hbm.at[idx_s.at[d]], rows.at[d], sem)

    def body(s, _):
        buf = lax.rem(s, NBUF)
        pltpu.make_async_copy(y_hbm.at[idx_s.at[s]], rows.at[buf], sem).wait()
        d1 = pltpu.async_copy(ones_v, hist_d.at[idx_d.at[s]], dsem, add=True)
        pltpu.sync_copy(rows.at[buf], agg.at[idx_d.at[s]], add=True)

        @pl.when(s + NBUF < STEPS)
        def _():
            pltpu.async_copy(y_hbm.at[idx_s.at[s + NBUF]], rows.at[buf], sem)
        d1.wait()
        return 0
    lax.fori_loop(0, STEPS, body, 0)

    plsc.subcore_barrier()
    pltpu.sync_copy(agg.at[pl.ds(base, ROWS_PER_TILE)], out_hbm.at[gid])
    pltpu.sync_copy(hist_d.at[pl.ds(base, ROWS_PER_TILE)], out_d.at[gid])


def _mm_body(x_ref, w_ref, dsrc_ref, y_ref):
    d = dsrc_ref[0, :, :1] + dsrc_ref[1, :, :1]
    norm = jnp.where(d > 0, lax.rsqrt(jnp.maximum(d, 1.0)), 0.0)
    y_ref[...] = jnp.dot(x_ref[...], w_ref[...],
                         preferred_element_type=jnp.float32) * norm


def _comb_body(agg_ref, ddst_ref, b_ref, h_ref):
    a = agg_ref[0] + agg_ref[1]
    d = ddst_ref[0, :, :1] + ddst_ref[1, :, :1]
    norm = jnp.where(d > 0, lax.rsqrt(jnp.maximum(d, 1.0)), 0.0)
    h_ref[...] = jnp.maximum(a * norm + b_ref[...], 0.0)


_BLK = 1024
_GRID = N_PAD // _BLK
_CBLK = 1000
_CGRID = N_NODES // _CBLK


def kernel(in_feat, edge_index, W, b):
    src = edge_index[0]
    dst = edge_index[1]
    pad_e = E_PAD - N_EDGES
    pad_idx = jnp.full((pad_e,), N_PAD - 1, jnp.int32)
    src3 = jnp.concatenate([src, pad_idx]).reshape(NW, STEPS, LANES)
    dst3 = jnp.concatenate([dst, pad_idx]).reshape(NW, STEPS, LANES)

    deg_s_r = _deg(src3)
    dsrc = deg_s_r.reshape(2, N_PAD, NUM_CLASSES)

    y = pl.pallas_call(
        _mm_body,
        grid=(_GRID,),
        in_specs=[
            pl.BlockSpec((_BLK, IN_FEATS), lambda i: (i, 0)),
            pl.BlockSpec((IN_FEATS, NUM_CLASSES), lambda i: (0, 0)),
            pl.BlockSpec((2, _BLK, NUM_CLASSES), lambda i: (0, i, 0)),
        ],
        out_specs=pl.BlockSpec((_BLK, NUM_CLASSES), lambda i: (i, 0)),
        out_shape=jax.ShapeDtypeStruct((N_PAD, NUM_CLASSES), jnp.float32),
    )(in_feat, W, dsrc)

    agg_r, deg_d_r = _agg(y, src3, dst3)
    agg3 = agg_r.reshape(2, N_PAD, NUM_CLASSES)
    ddst = deg_d_r.reshape(2, N_PAD, NUM_CLASSES)

    h = pl.pallas_call(
        _comb_body,
        grid=(_CGRID,),
        in_specs=[
            pl.BlockSpec((2, _CBLK, NUM_CLASSES), lambda i: (0, i, 0)),
            pl.BlockSpec((2, _CBLK, NUM_CLASSES), lambda i: (0, i, 0)),
            pl.BlockSpec((1, NUM_CLASSES), lambda i: (0, 0)),
        ],
        out_specs=pl.BlockSpec((_CBLK, NUM_CLASSES), lambda i: (i, 0)),
        out_shape=jax.ShapeDtypeStruct((N_NODES, NUM_CLASSES), jnp.float32),
    )(agg3, ddst, b.reshape(1, NUM_CLASSES))

    return h


# speedup vs baseline: 27.0313x; 1.0078x over previous
import functools

import jax
import jax.numpy as jnp
from jax import lax
from jax.experimental import pallas as pl
from jax.experimental.pallas import tpu as pltpu
from jax.experimental.pallas import tpu_sc as plsc

N_NODES = 10000
N_EDGES = 320000
IN_FEATS = 128
NUM_CLASSES = 16

N_PAD = 10240
NW = 32
STEPS = 80
LANES = 128
E_PER_TILE = STEPS * LANES
E_PAD = NW * E_PER_TILE
ROWS_PER_TILE = N_PAD // 16
NBUF = 16

_mesh = plsc.VectorSubcoreMesh(core_axis_name="c", subcore_axis_name="s")
_sc_params = pltpu.CompilerParams(use_tc_tiling_on_sc=False)


@functools.partial(
    pl.kernel,
    mesh=_mesh,
    out_type=jax.ShapeDtypeStruct((NW, ROWS_PER_TILE, NUM_CLASSES),
                                  jnp.float32),
    scratch_types=[
        pltpu.VMEM((STEPS, LANES), jnp.int32),
        pltpu.VMEM((LANES, NUM_CLASSES), jnp.float32),
        pltpu.VMEM((ROWS_PER_TILE, NUM_CLASSES), jnp.float32),
        pltpu.VMEM_SHARED((N_PAD, NUM_CLASSES), jnp.float32),
    ],
    compiler_params=_sc_params,
)
def _deg(src_hbm, out_s, idx_s, ones_v, zeros_v, hist_s):
    cid = lax.axis_index("c")
    sid = lax.axis_index("s")
    gid = cid * 16 + sid

    def fill(i, _):
        ones_v[i] = jnp.ones((16,), jnp.float32)
        zeros_v[i] = jnp.zeros((16,), jnp.float32)
        return 0
    lax.fori_loop(0, LANES, fill, 0)

    def fillz(i, _):
        zeros_v[i] = jnp.zeros((16,), jnp.float32)
        return 0
    lax.fori_loop(LANES, ROWS_PER_TILE, fillz, 0)

    base = pl.multiple_of(sid * ROWS_PER_TILE, 128)
    pltpu.sync_copy(zeros_v, hist_s.at[pl.ds(base, ROWS_PER_TILE)])
    pltpu.sync_copy(src_hbm.at[gid], idx_s)
    plsc.subcore_barrier()

    def body(s, _):
        pltpu.sync_copy(ones_v, hist_s.at[idx_s.at[s]], add=True)
        return 0
    lax.fori_loop(0, STEPS, body, 0)

    plsc.subcore_barrier()
    pltpu.sync_copy(hist_s.at[pl.ds(base, ROWS_PER_TILE)], out_s.at[gid])


@functools.partial(
    pl.kernel,
    mesh=_mesh,
    out_type=[
        jax.ShapeDtypeStruct((NW, ROWS_PER_TILE, NUM_CLASSES), jnp.float32),
        jax.ShapeDtypeStruct((NW, ROWS_PER_TILE, NUM_CLASSES), jnp.float32),
    ],
    scratch_types=[
        pltpu.VMEM((STEPS, LANES), jnp.int32),
        pltpu.VMEM((STEPS, LANES), jnp.int32),
        pltpu.VMEM((NBUF, LANES, NUM_CLASSES), jnp.float32),
        pltpu.VMEM((LANES, NUM_CLASSES), jnp.float32),
        pltpu.VMEM((ROWS_PER_TILE, NUM_CLASSES), jnp.float32),
        pltpu.VMEM_SHARED((N_PAD, NUM_CLASSES), jnp.float32),
        pltpu.VMEM_SHARED((N_PAD, NUM_CLASSES), jnp.float32),
        pltpu.SemaphoreType.DMA,
        pltpu.SemaphoreType.DMA,
    ],
    compiler_params=_sc_params,
)
def _agg(y_hbm, src_hbm, dst_hbm, out_hbm, out_d, idx_s, idx_d, rows,
         ones_v, zeros_v, agg, hist_d, sem, dsem):
    cid = lax.axis_index("c")
    sid = lax.axis_index("s")
    gid = cid * 16 + sid

    def fill(i, _):
        ones_v[i] = jnp.ones((16,), jnp.float32)
        zeros_v[i] = jnp.zeros((16,), jnp.float32)
        return 0
    lax.fori_loop(0, LANES, fill, 0)

    def fillz(i, _):
        zeros_v[i] = jnp.zeros((16,), jnp.float32)
        return 0
    lax.fori_loop(LANES, ROWS_PER_TILE, fillz, 0)

    base = pl.multiple_of(sid * ROWS_PER_TILE, 128)
    pltpu.sync_copy(zeros_v, agg.at[pl.ds(base, ROWS_PER_TILE)])
    pltpu.sync_copy(zeros_v, hist_d.at[pl.ds(base, ROWS_PER_TILE)])
    pltpu.sync_copy(src_hbm.at[gid], idx_s)
    pltpu.sync_copy(dst_hbm.at[gid], idx_d)
    plsc.subcore_barrier()

    for d in range(NBUF):
        pltpu.async_copy(y_hbm.at[idx_s.at[d]], rows.at[d], sem)

    def body(s, _):
        buf = lax.rem(s, NBUF)
        pltpu.make_async_copy(y_hbm.at[idx_s.at[s]], rows.at[buf], sem).wait()
        d1 = pltpu.async_copy(ones_v, hist_d.at[idx_d.at[s]], dsem, add=True)
        pltpu.sync_copy(rows.at[buf], agg.at[idx_d.at[s]], add=True)

        @pl.when(s + NBUF < STEPS)
        def _():
            pltpu.async_copy(y_hbm.at[idx_s.at[s + NBUF]], rows.at[buf], sem)
        d1.wait()
        return 0
    lax.fori_loop(0, STEPS, body, 0)

    plsc.subcore_barrier()
    pltpu.sync_copy(agg.at[pl.ds(base, ROWS_PER_TILE)], out_hbm.at[gid])
    pltpu.sync_copy(hist_d.at[pl.ds(base, ROWS_PER_TILE)], out_d.at[gid])


def _mm_body(x_ref, w_ref, dsrc_ref, y_ref):
    d = dsrc_ref[0, :, :1] + dsrc_ref[1, :, :1]
    norm = jnp.where(d > 0, lax.rsqrt(jnp.maximum(d, 1.0)), 0.0)
    y_ref[...] = jnp.dot(x_ref[...], w_ref[...],
                         preferred_element_type=jnp.float32) * norm


def _comb_body(agg_ref, ddst_ref, b_ref, h_ref):
    a = agg_ref[0] + agg_ref[1]
    d = ddst_ref[0, :, :1] + ddst_ref[1, :, :1]
    norm = jnp.where(d > 0, lax.rsqrt(jnp.maximum(d, 1.0)), 0.0)
    h_ref[...] = jnp.maximum(a * norm + b_ref[...], 0.0)


_BLK = 1024
_GRID = N_PAD // _BLK
_CBLK = 1000
_CGRID = N_NODES // _CBLK


def kernel(in_feat, edge_index, W, b):
    src = edge_index[0]
    dst = edge_index[1]
    pad_e = E_PAD - N_EDGES
    pad_idx = jnp.full((pad_e,), N_PAD - 1, jnp.int32)
    src3 = jnp.concatenate([src, pad_idx]).reshape(NW, STEPS, LANES)
    dst3 = jnp.concatenate([dst, pad_idx]).reshape(NW, STEPS, LANES)

    deg_s_r = _deg(src3)
    dsrc = deg_s_r.reshape(2, N_PAD, NUM_CLASSES)

    y = pl.pallas_call(
        _mm_body,
        grid=(_GRID,),
        in_specs=[
            pl.BlockSpec((_BLK, IN_FEATS), lambda i: (i, 0)),
            pl.BlockSpec((IN_FEATS, NUM_CLASSES), lambda i: (0, 0)),
            pl.BlockSpec((2, _BLK, NUM_CLASSES), lambda i: (0, i, 0)),
        ],
        out_specs=pl.BlockSpec((_BLK, NUM_CLASSES), lambda i: (i, 0)),
        out_shape=jax.ShapeDtypeStruct((N_PAD, NUM_CLASSES), jnp.float32),
    )(in_feat, W, dsrc)

    agg_r, deg_d_r = _agg(y, src3, dst3)
    agg3 = agg_r.reshape(2, N_PAD, NUM_CLASSES)
    ddst = deg_d_r.reshape(2, N_PAD, NUM_CLASSES)

    h = pl.pallas_call(
        _comb_body,
        grid=(_CGRID,),
        in_specs=[
            pl.BlockSpec((2, _CBLK, NUM_CLASSES), lambda i: (0, i, 0)),
            pl.BlockSpec((2, _CBLK, NUM_CLASSES), lambda i: (0, i, 0)),
            pl.BlockSpec((1, NUM_CLASSES), lambda i: (0, 0)),
        ],
        out_specs=pl.BlockSpec((_CBLK, NUM_CLASSES), lambda i: (i, 0)),
        out_shape=jax.ShapeDtypeStruct((N_NODES, NUM_CLASSES), jnp.float32),
    )(agg3, ddst, b.reshape(1, NUM_CLASSES))

    return h


# speedup vs baseline: 37.6788x; 1.3939x over previous
import functools

import jax
import jax.numpy as jnp
from jax import lax
from jax.experimental import pallas as pl
from jax.experimental.pallas import tpu as pltpu
from jax.experimental.pallas import tpu_sc as plsc

N_NODES = 10000
N_EDGES = 320000
IN_FEATS = 128
NUM_CLASSES = 16

N_PAD = 10240
NW = 32
LANES = 128
E_ROWS = N_EDGES // LANES
BASE_STEPS = E_ROWS // NW
TAIL_ROWS = E_ROWS - BASE_STEPS * NW
ROWS_PER_TILE = N_PAD // 16
NBUF = 16

_mesh = plsc.VectorSubcoreMesh(core_axis_name="c", subcore_axis_name="s")
_sc_params = pltpu.CompilerParams(use_tc_tiling_on_sc=False)

_PART = jax.ShapeDtypeStruct((16, ROWS_PER_TILE, NUM_CLASSES), jnp.float32)


def _load_idx(edge_hbm, which, gid, idx):
    pltpu.sync_copy(edge_hbm.at[which, pl.ds(gid * BASE_STEPS, BASE_STEPS)],
                    idx.at[pl.ds(0, BASE_STEPS)])

    @pl.when(gid < TAIL_ROWS)
    def _():
        pltpu.sync_copy(
            edge_hbm.at[which, pl.ds(NW * BASE_STEPS + gid, 1)],
            idx.at[pl.ds(BASE_STEPS, 1)])


@functools.partial(
    pl.kernel,
    mesh=_mesh,
    out_type=[_PART, _PART],
    scratch_types=[
        pltpu.VMEM((BASE_STEPS + 1, LANES), jnp.int32),
        pltpu.VMEM((LANES, NUM_CLASSES), jnp.float32),
        pltpu.VMEM((ROWS_PER_TILE, NUM_CLASSES), jnp.float32),
        pltpu.VMEM_SHARED((N_PAD, NUM_CLASSES), jnp.float32),
    ],
    compiler_params=_sc_params,
)
def _deg(edge_hbm, out_s0, out_s1, idx_s, ones_v, zeros_v, hist_s):
    cid = lax.axis_index("c")
    sid = lax.axis_index("s")
    gid = cid * 16 + sid
    nsteps = BASE_STEPS + jnp.where(gid < TAIL_ROWS, 1, 0)

    def fill(i, _):
        ones_v[i] = jnp.ones((16,), jnp.float32)
        zeros_v[i] = jnp.zeros((16,), jnp.float32)
        return 0
    lax.fori_loop(0, LANES, fill, 0)

    def fillz(i, _):
        zeros_v[i] = jnp.zeros((16,), jnp.float32)
        return 0
    lax.fori_loop(LANES, ROWS_PER_TILE, fillz, 0)

    base = pl.multiple_of(sid * ROWS_PER_TILE, 128)
    pltpu.sync_copy(zeros_v, hist_s.at[pl.ds(base, ROWS_PER_TILE)])
    _load_idx(edge_hbm, 0, gid, idx_s)
    plsc.subcore_barrier()

    def body(s, _):
        pltpu.sync_copy(ones_v, hist_s.at[idx_s.at[s]], add=True)
        return 0
    lax.fori_loop(0, nsteps, body, 0)

    plsc.subcore_barrier()

    @pl.when(cid == 0)
    def _():
        pltpu.sync_copy(hist_s.at[pl.ds(base, ROWS_PER_TILE)], out_s0.at[sid])

    @pl.when(cid == 1)
    def _():
        pltpu.sync_copy(hist_s.at[pl.ds(base, ROWS_PER_TILE)], out_s1.at[sid])


@functools.partial(
    pl.kernel,
    mesh=_mesh,
    out_type=[_PART, _PART, _PART, _PART],
    scratch_types=[
        pltpu.VMEM((BASE_STEPS + 1, LANES), jnp.int32),
        pltpu.VMEM((BASE_STEPS + 1, LANES), jnp.int32),
        pltpu.VMEM((NBUF, LANES, NUM_CLASSES), jnp.float32),
        pltpu.VMEM((LANES, NUM_CLASSES), jnp.float32),
        pltpu.VMEM((ROWS_PER_TILE, NUM_CLASSES), jnp.float32),
        pltpu.VMEM_SHARED((N_PAD, NUM_CLASSES), jnp.float32),
        pltpu.VMEM_SHARED((N_PAD, NUM_CLASSES), jnp.float32),
        pltpu.SemaphoreType.DMA,
        pltpu.SemaphoreType.DMA,
    ],
    compiler_params=_sc_params,
)
def _agg(y_hbm, edge_hbm, out_a0, out_a1, out_d0, out_d1, idx_s, idx_d, rows,
         ones_v, zeros_v, agg, hist_d, sem, dsem):
    cid = lax.axis_index("c")
    sid = lax.axis_index("s")
    gid = cid * 16 + sid
    nsteps = BASE_STEPS + jnp.where(gid < TAIL_ROWS, 1, 0)

    def fill(i, _):
        ones_v[i] = jnp.ones((16,), jnp.float32)
        zeros_v[i] = jnp.zeros((16,), jnp.float32)
        return 0
    lax.fori_loop(0, LANES, fill, 0)

    def fillz(i, _):
        zeros_v[i] = jnp.zeros((16,), jnp.float32)
        return 0
    lax.fori_loop(LANES, ROWS_PER_TILE, fillz, 0)

    base = pl.multiple_of(sid * ROWS_PER_TILE, 128)
    pltpu.sync_copy(zeros_v, agg.at[pl.ds(base, ROWS_PER_TILE)])
    pltpu.sync_copy(zeros_v, hist_d.at[pl.ds(base, ROWS_PER_TILE)])
    _load_idx(edge_hbm, 0, gid, idx_s)
    _load_idx(edge_hbm, 1, gid, idx_d)
    plsc.subcore_barrier()

    for d in range(NBUF):
        pltpu.async_copy(y_hbm.at[idx_s.at[d]], rows.at[d], sem)

    def body(s, _):
        buf = lax.rem(s, NBUF)
        pltpu.make_async_copy(y_hbm.at[idx_s.at[s]], rows.at[buf], sem).wait()
        d1 = pltpu.async_copy(ones_v, hist_d.at[idx_d.at[s]], dsem, add=True)
        pltpu.sync_copy(rows.at[buf], agg.at[idx_d.at[s]], add=True)

        @pl.when(s + NBUF < nsteps)
        def _():
            pltpu.async_copy(y_hbm.at[idx_s.at[s + NBUF]], rows.at[buf], sem)
        d1.wait()
        return 0
    lax.fori_loop(0, nsteps, body, 0)

    plsc.subcore_barrier()

    @pl.when(cid == 0)
    def _():
        pltpu.sync_copy(agg.at[pl.ds(base, ROWS_PER_TILE)], out_a0.at[sid])
        pltpu.sync_copy(hist_d.at[pl.ds(base, ROWS_PER_TILE)], out_d0.at[sid])

    @pl.when(cid == 1)
    def _():
        pltpu.sync_copy(agg.at[pl.ds(base, ROWS_PER_TILE)], out_a1.at[sid])
        pltpu.sync_copy(hist_d.at[pl.ds(base, ROWS_PER_TILE)], out_d1.at[sid])


def _norm_of(p0_ref, p1_ref):
    d = p0_ref[0, :, :1] + p1_ref[0, :, :1]
    return jnp.where(d > 0, lax.rsqrt(jnp.maximum(d, 1.0)), 0.0)


def _mm_body(x_ref, w_ref, ds0_ref, ds1_ref, y_ref):
    y_ref[...] = jnp.dot(x_ref[...], w_ref[...],
                         preferred_element_type=jnp.float32) * _norm_of(
                             ds0_ref, ds1_ref)


def _comb_body(a0_ref, a1_ref, d0_ref, d1_ref, b_ref, h_ref):
    a = a0_ref[0] + a1_ref[0]
    h_ref[...] = jnp.maximum(
        a * _norm_of(d0_ref, d1_ref) + b_ref[...], 0.0)


_BLK = ROWS_PER_TILE
_GRID = N_PAD // _BLK

_part_spec = pl.BlockSpec((1, _BLK, NUM_CLASSES), lambda i: (i, 0, 0))


def kernel(in_feat, edge_index, W, b):
    edge3 = edge_index.reshape(2, E_ROWS, LANES)

    ds0, ds1 = _deg(edge3)

    y = pl.pallas_call(
        _mm_body,
        grid=(_GRID,),
        in_specs=[
            pl.BlockSpec((_BLK, IN_FEATS), lambda i: (i, 0)),
            pl.BlockSpec((IN_FEATS, NUM_CLASSES), lambda i: (0, 0)),
            _part_spec,
            _part_spec,
        ],
        out_specs=pl.BlockSpec((_BLK, NUM_CLASSES), lambda i: (i, 0)),
        out_shape=jax.ShapeDtypeStruct((N_PAD, NUM_CLASSES), jnp.float32),
    )(in_feat, W, ds0, ds1)

    a0, a1, d0, d1 = _agg(y, edge3)

    h = pl.pallas_call(
        _comb_body,
        grid=(_GRID,),
        in_specs=[_part_spec, _part_spec, _part_spec, _part_spec,
                  pl.BlockSpec((1, NUM_CLASSES), lambda i: (0, 0))],
        out_specs=pl.BlockSpec((_BLK, NUM_CLASSES), lambda i: (i, 0)),
        out_shape=jax.ShapeDtypeStruct((N_NODES, NUM_CLASSES), jnp.float32),
    )(a0, a1, d0, d1, b.reshape(1, NUM_CLASSES))

    return h


# speedup vs baseline: 44.3865x; 1.1780x over previous
import functools

import jax
import jax.numpy as jnp
from jax import lax
from jax.experimental import pallas as pl
from jax.experimental.pallas import tpu as pltpu
from jax.experimental.pallas import tpu_sc as plsc

N_NODES = 10000
N_EDGES = 320000
IN_FEATS = 128
NUM_CLASSES = 16

N_PAD = 10240
NW = 32
LANES = 128
E_ROWS = N_EDGES // LANES
BASE_STEPS = E_ROWS // NW
TAIL_ROWS = E_ROWS - BASE_STEPS * NW
ROWS_PER_TILE = N_PAD // 16
NODES_PER_TILE = N_PAD // NW
NBUF = 16

_mesh = plsc.VectorSubcoreMesh(core_axis_name="c", subcore_axis_name="s")
_sc_params = pltpu.CompilerParams(use_tc_tiling_on_sc=False)

_PART = jax.ShapeDtypeStruct((N_PAD, NUM_CLASSES), jnp.float32)


def _wid(cid, sid):
    return cid * 16 + sid


def _load_idx(edge_hbm, which, gid, idx):
    pltpu.sync_copy(edge_hbm.at[which, pl.ds(gid * BASE_STEPS, BASE_STEPS)],
                    idx.at[pl.ds(0, BASE_STEPS)])

    @pl.when(gid < TAIL_ROWS)
    def _():
        pltpu.sync_copy(
            edge_hbm.at[which, pl.ds(NW * BASE_STEPS + gid, 1)],
            idx.at[pl.ds(BASE_STEPS, 1)])


def _rsqrt_norm(d16):
    x = jnp.maximum(d16, 1.0)
    t = x
    r = jnp.full((16,), 1.0, jnp.float32)
    for j in range(9, 0, -1):
        c = t >= float(4 ** j)
        t = jnp.where(c, t * float(4.0 ** (-j)), t)
        r = jnp.where(c, r * float(2.0 ** (-j)), r)
    c = t >= 2.0
    t = jnp.where(c, t * 0.5, t)
    r = jnp.where(c, r * 0.7071067811865476, r)
    s = jnp.full((16,), 0.85, jnp.float32)
    for _ in range(4):
        s = s * (1.5 - 0.5 * t * s * s)
    return jnp.where(d16 > 0, s * r, 0.0)


@functools.partial(
    pl.kernel,
    mesh=_mesh,
    out_type=[_PART, _PART],
    scratch_types=[
        pltpu.VMEM((BASE_STEPS + 1, LANES), jnp.int32),
        pltpu.VMEM((LANES, NUM_CLASSES), jnp.float32),
        pltpu.VMEM((ROWS_PER_TILE, NUM_CLASSES), jnp.float32),
        pltpu.VMEM_SHARED((N_PAD, NUM_CLASSES), jnp.float32),
    ],
    compiler_params=_sc_params,
)
def _deg(edge_hbm, out_s0, out_s1, idx_s, ones_v, zeros_v, hist_s):
    cid = lax.axis_index("c")
    sid = lax.axis_index("s")
    gid = _wid(cid, sid)
    nsteps = BASE_STEPS + jnp.where(gid < TAIL_ROWS, 1, 0)

    def fill(i, _):
        ones_v[i] = jnp.ones((16,), jnp.float32)
        zeros_v[i] = jnp.zeros((16,), jnp.float32)
        return 0
    lax.fori_loop(0, LANES, fill, 0)

    def fillz(i, _):
        zeros_v[i] = jnp.zeros((16,), jnp.float32)
        return 0
    lax.fori_loop(LANES, ROWS_PER_TILE, fillz, 0)

    base = pl.multiple_of(sid * ROWS_PER_TILE, 128)
    pltpu.sync_copy(zeros_v, hist_s.at[pl.ds(base, ROWS_PER_TILE)])
    _load_idx(edge_hbm, 0, gid, idx_s)
    plsc.subcore_barrier()

    def body(s, _):
        pltpu.sync_copy(ones_v, hist_s.at[idx_s.at[s]], add=True)
        return 0
    lax.fori_loop(0, nsteps, body, 0)

    plsc.subcore_barrier()

    @pl.when(cid == 0)
    def _():
        pltpu.sync_copy(hist_s.at[pl.ds(base, ROWS_PER_TILE)],
                        out_s0.at[pl.ds(base, ROWS_PER_TILE)])

    @pl.when(cid == 1)
    def _():
        pltpu.sync_copy(hist_s.at[pl.ds(base, ROWS_PER_TILE)],
                        out_s1.at[pl.ds(base, ROWS_PER_TILE)])


def _mm_body(x_ref, w_ref, y_ref):
    xw = jnp.dot(x_ref[...], w_ref[...], preferred_element_type=jnp.float32)
    y_ref[...] = jnp.concatenate(
        [xw, jnp.zeros((xw.shape[0], LANES - NUM_CLASSES), jnp.float32)],
        axis=1)


@functools.partial(
    pl.kernel,
    mesh=_mesh,
    out_type=_PART,
    scratch_types=[
        pltpu.VMEM((NODES_PER_TILE, LANES), jnp.float32),
        pltpu.VMEM((NODES_PER_TILE, NUM_CLASSES), jnp.float32),
        pltpu.VMEM((NODES_PER_TILE, NUM_CLASSES), jnp.float32),
        pltpu.VMEM((NODES_PER_TILE, NUM_CLASSES), jnp.float32),
    ],
    compiler_params=_sc_params,
)
def _scale(xw_hbm, ds0_hbm, ds1_hbm, y_hbm, xv, d0v, d1v, yv):
    cid = lax.axis_index("c")
    sid = lax.axis_index("s")
    gid = _wid(cid, sid)
    start = pl.multiple_of(gid * NODES_PER_TILE, 128)
    pltpu.sync_copy(xw_hbm.at[pl.ds(start, NODES_PER_TILE)], xv)
    pltpu.sync_copy(ds0_hbm.at[pl.ds(start, NODES_PER_TILE)], d0v)
    pltpu.sync_copy(ds1_hbm.at[pl.ds(start, NODES_PER_TILE)], d1v)

    def body(i, _):
        norm = _rsqrt_norm(d0v[i] + d1v[i])
        yv[i] = xv[i, pl.ds(0, NUM_CLASSES)] * norm
        return 0
    lax.fori_loop(0, NODES_PER_TILE, body, 0)

    pltpu.sync_copy(yv, y_hbm.at[pl.ds(start, NODES_PER_TILE)])


@functools.partial(
    pl.kernel,
    mesh=_mesh,
    out_type=[_PART, _PART, _PART, _PART],
    scratch_types=[
        pltpu.VMEM((BASE_STEPS + 1, LANES), jnp.int32),
        pltpu.VMEM((BASE_STEPS + 1, LANES), jnp.int32),
        pltpu.VMEM((NBUF, LANES, NUM_CLASSES), jnp.float32),
        pltpu.VMEM((LANES, NUM_CLASSES), jnp.float32),
        pltpu.VMEM((ROWS_PER_TILE, NUM_CLASSES), jnp.float32),
        pltpu.VMEM_SHARED((N_PAD, NUM_CLASSES), jnp.float32),
        pltpu.VMEM_SHARED((N_PAD, NUM_CLASSES), jnp.float32),
        pltpu.SemaphoreType.DMA,
        pltpu.SemaphoreType.DMA,
    ],
    compiler_params=_sc_params,
)
def _agg(y_hbm, edge_hbm, out_a0, out_a1, out_d0, out_d1, idx_s, idx_d, rows,
         ones_v, zeros_v, agg, hist_d, sem, dsem):
    cid = lax.axis_index("c")
    sid = lax.axis_index("s")
    gid = _wid(cid, sid)
    nsteps = BASE_STEPS + jnp.where(gid < TAIL_ROWS, 1, 0)

    def fill(i, _):
        ones_v[i] = jnp.ones((16,), jnp.float32)
        zeros_v[i] = jnp.zeros((16,), jnp.float32)
        return 0
    lax.fori_loop(0, LANES, fill, 0)

    def fillz(i, _):
        zeros_v[i] = jnp.zeros((16,), jnp.float32)
        return 0
    lax.fori_loop(LANES, ROWS_PER_TILE, fillz, 0)

    base = pl.multiple_of(sid * ROWS_PER_TILE, 128)
    pltpu.sync_copy(zeros_v, agg.at[pl.ds(base, ROWS_PER_TILE)])
    pltpu.sync_copy(zeros_v, hist_d.at[pl.ds(base, ROWS_PER_TILE)])
    _load_idx(edge_hbm, 0, gid, idx_s)
    _load_idx(edge_hbm, 1, gid, idx_d)
    plsc.subcore_barrier()

    for d in range(NBUF):
        pltpu.async_copy(y_hbm.at[idx_s.at[d]], rows.at[d], sem)

    def body(s, _):
        buf = lax.rem(s, NBUF)
        pltpu.make_async_copy(y_hbm.at[idx_s.at[s]], rows.at[buf], sem).wait()
        d1 = pltpu.async_copy(ones_v, hist_d.at[idx_d.at[s]], dsem, add=True)
        pltpu.sync_copy(rows.at[buf], agg.at[idx_d.at[s]], add=True)

        @pl.when(s + NBUF < nsteps)
        def _():
            pltpu.async_copy(y_hbm.at[idx_s.at[s + NBUF]], rows.at[buf], sem)
        d1.wait()
        return 0
    lax.fori_loop(0, nsteps, body, 0)

    plsc.subcore_barrier()

    @pl.when(cid == 0)
    def _():
        pltpu.sync_copy(agg.at[pl.ds(base, ROWS_PER_TILE)],
                        out_a0.at[pl.ds(base, ROWS_PER_TILE)])
        pltpu.sync_copy(hist_d.at[pl.ds(base, ROWS_PER_TILE)],
                        out_d0.at[pl.ds(base, ROWS_PER_TILE)])

    @pl.when(cid == 1)
    def _():
        pltpu.sync_copy(agg.at[pl.ds(base, ROWS_PER_TILE)],
                        out_a1.at[pl.ds(base, ROWS_PER_TILE)])
        pltpu.sync_copy(hist_d.at[pl.ds(base, ROWS_PER_TILE)],
                        out_d1.at[pl.ds(base, ROWS_PER_TILE)])


@functools.partial(
    pl.kernel,
    mesh=_mesh,
    out_type=jax.ShapeDtypeStruct((N_NODES, NUM_CLASSES), jnp.float32),
    scratch_types=[
        pltpu.VMEM((NODES_PER_TILE, NUM_CLASSES), jnp.float32),
        pltpu.VMEM((NODES_PER_TILE, NUM_CLASSES), jnp.float32),
        pltpu.VMEM((NODES_PER_TILE, NUM_CLASSES), jnp.float32),
        pltpu.VMEM((NODES_PER_TILE, NUM_CLASSES), jnp.float32),
        pltpu.VMEM((NODES_PER_TILE, NUM_CLASSES), jnp.float32),
        pltpu.VMEM((NUM_CLASSES,), jnp.float32),
    ],
    compiler_params=_sc_params,
)
def _comb(a0_hbm, a1_hbm, d0_hbm, d1_hbm, b_hbm, h_hbm,
          a0v, a1v, d0v, d1v, hv, bv):
    cid = lax.axis_index("c")
    sid = lax.axis_index("s")
    gid = _wid(cid, sid)
    start = pl.multiple_of(gid * NODES_PER_TILE, 128)
    pltpu.sync_copy(a0_hbm.at[pl.ds(start, NODES_PER_TILE)], a0v)
    pltpu.sync_copy(a1_hbm.at[pl.ds(start, NODES_PER_TILE)], a1v)
    pltpu.sync_copy(d0_hbm.at[pl.ds(start, NODES_PER_TILE)], d0v)
    pltpu.sync_copy(d1_hbm.at[pl.ds(start, NODES_PER_TILE)], d1v)
    pltpu.sync_copy(b_hbm, bv)
    b16 = bv[...]

    def body(i, _):
        norm = _rsqrt_norm(d0v[i] + d1v[i])
        hv[i] = jnp.maximum((a0v[i] + a1v[i]) * norm + b16, 0.0)
        return 0
    lax.fori_loop(0, NODES_PER_TILE, body, 0)

    last = NW - 1
    tail = N_NODES - last * NODES_PER_TILE

    @pl.when(gid < last)
    def _():
        pltpu.sync_copy(hv, h_hbm.at[pl.ds(start, NODES_PER_TILE)])

    @pl.when(gid == last)
    def _():
        pltpu.sync_copy(hv.at[pl.ds(0, tail)],
                        h_hbm.at[pl.ds(last * NODES_PER_TILE, tail)])


_BLK = 1280
_GRID = N_PAD // _BLK


def kernel(in_feat, edge_index, W, b):
    edge3 = edge_index.reshape(2, E_ROWS, LANES)

    ds0, ds1 = _deg(edge3)

    xw = pl.pallas_call(
        _mm_body,
        grid=(_GRID,),
        in_specs=[
            pl.BlockSpec((_BLK, IN_FEATS), lambda i: (i, 0)),
            pl.BlockSpec((IN_FEATS, NUM_CLASSES), lambda i: (0, 0)),
        ],
        out_specs=pl.BlockSpec((_BLK, LANES), lambda i: (i, 0)),
        out_shape=jax.ShapeDtypeStruct((N_PAD, LANES), jnp.float32),
    )(in_feat, W)

    y = _scale(xw, ds0, ds1)
    a0, a1, d0, d1 = _agg(y, edge3)
    degd = d0 + d1
    normd = jnp.where(degd > 0, lax.rsqrt(jnp.maximum(degd, 1.0)), 0.0)
    h = jnp.maximum((a0 + a1) * normd + b[None, :], 0.0)
    return h[:N_NODES]


# speedup vs baseline: 45.9969x; 1.0363x over previous
import functools

import jax
import jax.numpy as jnp
from jax import lax
from jax.experimental import pallas as pl
from jax.experimental.pallas import tpu as pltpu
from jax.experimental.pallas import tpu_sc as plsc

N_NODES = 10000
N_EDGES = 320000
IN_FEATS = 128
NUM_CLASSES = 16

N_PAD = 10240
NW = 32
LANES = 128
E_ROWS = N_EDGES // LANES
BASE_STEPS = E_ROWS // NW
TAIL_ROWS = E_ROWS - BASE_STEPS * NW
ROWS_PER_TILE = N_PAD // 16
NODES_PER_TILE = N_PAD // NW
NBUF = 16

_mesh = plsc.VectorSubcoreMesh(core_axis_name="c", subcore_axis_name="s")
_sc_params = pltpu.CompilerParams(use_tc_tiling_on_sc=False)

_PART = jax.ShapeDtypeStruct((N_PAD, NUM_CLASSES), jnp.float32)


def _wid(cid, sid):
    return cid * 16 + sid


def _load_idx(edge_hbm, which, gid, idx):
    pltpu.sync_copy(edge_hbm.at[which, pl.ds(gid * BASE_STEPS, BASE_STEPS)],
                    idx.at[pl.ds(0, BASE_STEPS)])

    @pl.when(gid < TAIL_ROWS)
    def _():
        pltpu.sync_copy(
            edge_hbm.at[which, pl.ds(NW * BASE_STEPS + gid, 1)],
            idx.at[pl.ds(BASE_STEPS, 1)])


def _rsqrt_norm(d16):
    x = jnp.maximum(d16, 1.0)
    t = x
    r = jnp.full((16,), 1.0, jnp.float32)
    for j in range(9, 0, -1):
        c = t >= float(4 ** j)
        t = jnp.where(c, t * float(4.0 ** (-j)), t)
        r = jnp.where(c, r * float(2.0 ** (-j)), r)
    c = t >= 2.0
    t = jnp.where(c, t * 0.5, t)
    r = jnp.where(c, r * 0.7071067811865476, r)
    s = jnp.full((16,), 0.85, jnp.float32)
    for _ in range(4):
        s = s * (1.5 - 0.5 * t * s * s)
    return jnp.where(d16 > 0, s * r, 0.0)


@functools.partial(
    pl.kernel,
    mesh=_mesh,
    out_type=[_PART, _PART],
    scratch_types=[
        pltpu.VMEM((BASE_STEPS + 1, LANES), jnp.int32),
        pltpu.VMEM((LANES, NUM_CLASSES), jnp.float32),
        pltpu.VMEM((ROWS_PER_TILE, NUM_CLASSES), jnp.float32),
        pltpu.VMEM_SHARED((N_PAD, NUM_CLASSES), jnp.float32),
    ],
    compiler_params=_sc_params,
)
def _deg(edge_hbm, out_s0, out_s1, idx_s, ones_v, zeros_v, hist_s):
    cid = lax.axis_index("c")
    sid = lax.axis_index("s")
    gid = _wid(cid, sid)
    nsteps = BASE_STEPS + jnp.where(gid < TAIL_ROWS, 1, 0)

    def fill(i, _):
        ones_v[i] = jnp.ones((16,), jnp.float32)
        zeros_v[i] = jnp.zeros((16,), jnp.float32)
        return 0
    lax.fori_loop(0, LANES, fill, 0)

    def fillz(i, _):
        zeros_v[i] = jnp.zeros((16,), jnp.float32)
        return 0
    lax.fori_loop(LANES, ROWS_PER_TILE, fillz, 0)

    base = pl.multiple_of(sid * ROWS_PER_TILE, 128)
    pltpu.sync_copy(zeros_v, hist_s.at[pl.ds(base, ROWS_PER_TILE)])
    _load_idx(edge_hbm, 0, gid, idx_s)
    plsc.subcore_barrier()

    def body(s, _):
        pltpu.sync_copy(ones_v, hist_s.at[idx_s.at[s]], add=True)
        return 0
    lax.fori_loop(0, nsteps, body, 0)

    plsc.subcore_barrier()

    @pl.when(cid == 0)
    def _():
        pltpu.sync_copy(hist_s.at[pl.ds(base, ROWS_PER_TILE)],
                        out_s0.at[pl.ds(base, ROWS_PER_TILE)])

    @pl.when(cid == 1)
    def _():
        pltpu.sync_copy(hist_s.at[pl.ds(base, ROWS_PER_TILE)],
                        out_s1.at[pl.ds(base, ROWS_PER_TILE)])


def _mm_body(x_ref, w_ref, y_ref):
    xw = jnp.dot(x_ref[...], w_ref[...], preferred_element_type=jnp.float32)
    y_ref[...] = jnp.concatenate(
        [xw, jnp.zeros((xw.shape[0], LANES - NUM_CLASSES), jnp.float32)],
        axis=1)


@functools.partial(
    pl.kernel,
    mesh=_mesh,
    out_type=_PART,
    scratch_types=[
        pltpu.VMEM((NODES_PER_TILE, LANES), jnp.float32),
        pltpu.VMEM((NODES_PER_TILE, NUM_CLASSES), jnp.float32),
        pltpu.VMEM((NODES_PER_TILE, NUM_CLASSES), jnp.float32),
        pltpu.VMEM((NODES_PER_TILE, NUM_CLASSES), jnp.float32),
    ],
    compiler_params=_sc_params,
)
def _scale(xw_hbm, ds0_hbm, ds1_hbm, y_hbm, xv, d0v, d1v, yv):
    cid = lax.axis_index("c")
    sid = lax.axis_index("s")
    gid = _wid(cid, sid)
    start = pl.multiple_of(gid * NODES_PER_TILE, 128)
    pltpu.sync_copy(xw_hbm.at[pl.ds(start, NODES_PER_TILE)], xv)
    pltpu.sync_copy(ds0_hbm.at[pl.ds(start, NODES_PER_TILE)], d0v)
    pltpu.sync_copy(ds1_hbm.at[pl.ds(start, NODES_PER_TILE)], d1v)

    def body(i, _):
        norm = _rsqrt_norm(d0v[i] + d1v[i])
        yv[i] = xv[i, pl.ds(0, NUM_CLASSES)] * norm
        return 0
    lax.fori_loop(0, NODES_PER_TILE, body, 0)

    pltpu.sync_copy(yv, y_hbm.at[pl.ds(start, NODES_PER_TILE)])


@functools.partial(
    pl.kernel,
    mesh=_mesh,
    out_type=[_PART, _PART, _PART, _PART],
    scratch_types=[
        pltpu.VMEM((BASE_STEPS + 1, LANES), jnp.int32),
        pltpu.VMEM((BASE_STEPS + 1, LANES), jnp.int32),
        pltpu.VMEM((NBUF, LANES, NUM_CLASSES), jnp.float32),
        pltpu.VMEM((LANES, NUM_CLASSES), jnp.float32),
        pltpu.VMEM((ROWS_PER_TILE, NUM_CLASSES), jnp.float32),
        pltpu.VMEM_SHARED((N_PAD, NUM_CLASSES), jnp.float32),
        pltpu.VMEM_SHARED((N_PAD, NUM_CLASSES), jnp.float32),
        pltpu.SemaphoreType.DMA,
        pltpu.SemaphoreType.DMA,
    ],
    compiler_params=_sc_params,
)
def _agg(y_hbm, edge_hbm, out_a0, out_a1, out_d0, out_d1, idx_s, idx_d, rows,
         ones_v, zeros_v, agg, hist_d, sem, dsem):
    cid = lax.axis_index("c")
    sid = lax.axis_index("s")
    gid = _wid(cid, sid)
    nsteps = BASE_STEPS + jnp.where(gid < TAIL_ROWS, 1, 0)

    def fill(i, _):
        ones_v[i] = jnp.ones((16,), jnp.float32)
        zeros_v[i] = jnp.zeros((16,), jnp.float32)
        return 0
    lax.fori_loop(0, LANES, fill, 0)

    def fillz(i, _):
        zeros_v[i] = jnp.zeros((16,), jnp.float32)
        return 0
    lax.fori_loop(LANES, ROWS_PER_TILE, fillz, 0)

    base = pl.multiple_of(sid * ROWS_PER_TILE, 128)
    pltpu.sync_copy(zeros_v, agg.at[pl.ds(base, ROWS_PER_TILE)])
    pltpu.sync_copy(zeros_v, hist_d.at[pl.ds(base, ROWS_PER_TILE)])
    _load_idx(edge_hbm, 0, gid, idx_s)
    _load_idx(edge_hbm, 1, gid, idx_d)
    plsc.subcore_barrier()

    for d in range(NBUF):
        pltpu.async_copy(y_hbm.at[idx_s.at[d]], rows.at[d], sem)

    def body(s, _):
        buf = lax.rem(s, NBUF)
        pltpu.make_async_copy(y_hbm.at[idx_s.at[s]], rows.at[buf], sem).wait()
        d1 = pltpu.async_copy(ones_v, hist_d.at[idx_d.at[s]], dsem, add=True)
        pltpu.sync_copy(rows.at[buf], agg.at[idx_d.at[s]], add=True)

        @pl.when(s + NBUF < nsteps)
        def _():
            pltpu.async_copy(y_hbm.at[idx_s.at[s + NBUF]], rows.at[buf], sem)
        d1.wait()
        return 0
    lax.fori_loop(0, nsteps, body, 0)

    plsc.subcore_barrier()

    @pl.when(cid == 0)
    def _():
        pltpu.sync_copy(agg.at[pl.ds(base, ROWS_PER_TILE)],
                        out_a0.at[pl.ds(base, ROWS_PER_TILE)])
        pltpu.sync_copy(hist_d.at[pl.ds(base, ROWS_PER_TILE)],
                        out_d0.at[pl.ds(base, ROWS_PER_TILE)])

    @pl.when(cid == 1)
    def _():
        pltpu.sync_copy(agg.at[pl.ds(base, ROWS_PER_TILE)],
                        out_a1.at[pl.ds(base, ROWS_PER_TILE)])
        pltpu.sync_copy(hist_d.at[pl.ds(base, ROWS_PER_TILE)],
                        out_d1.at[pl.ds(base, ROWS_PER_TILE)])


@functools.partial(
    pl.kernel,
    mesh=_mesh,
    out_type=jax.ShapeDtypeStruct((N_PAD, NUM_CLASSES), jnp.float32),
    scratch_types=[
        pltpu.VMEM((NODES_PER_TILE, NUM_CLASSES), jnp.float32),
        pltpu.VMEM((NODES_PER_TILE, NUM_CLASSES), jnp.float32),
        pltpu.VMEM((NODES_PER_TILE, NUM_CLASSES), jnp.float32),
        pltpu.VMEM((NODES_PER_TILE, NUM_CLASSES), jnp.float32),
        pltpu.VMEM((NODES_PER_TILE, NUM_CLASSES), jnp.float32),
        pltpu.VMEM((NUM_CLASSES,), jnp.float32),
    ],
    compiler_params=_sc_params,
)
def _comb(a0_hbm, a1_hbm, d0_hbm, d1_hbm, b_hbm, h_hbm,
          a0v, a1v, d0v, d1v, hv, bv):
    cid = lax.axis_index("c")
    sid = lax.axis_index("s")
    gid = _wid(cid, sid)
    start = pl.multiple_of(gid * NODES_PER_TILE, 128)
    pltpu.sync_copy(a0_hbm.at[pl.ds(start, NODES_PER_TILE)], a0v)
    pltpu.sync_copy(a1_hbm.at[pl.ds(start, NODES_PER_TILE)], a1v)
    pltpu.sync_copy(d0_hbm.at[pl.ds(start, NODES_PER_TILE)], d0v)
    pltpu.sync_copy(d1_hbm.at[pl.ds(start, NODES_PER_TILE)], d1v)
    pltpu.sync_copy(b_hbm, bv)
    b16 = bv[...]

    def body(i, _):
        norm = _rsqrt_norm(d0v[i] + d1v[i])
        hv[i] = jnp.maximum((a0v[i] + a1v[i]) * norm + b16, 0.0)
        return 0
    lax.fori_loop(0, NODES_PER_TILE, body, 0)

    pltpu.sync_copy(hv, h_hbm.at[pl.ds(start, NODES_PER_TILE)])


_BLK = 1280
_GRID = N_PAD // _BLK


def kernel(in_feat, edge_index, W, b):
    edge3 = edge_index.reshape(2, E_ROWS, LANES)

    ds0, ds1 = _deg(edge3)

    xw = pl.pallas_call(
        _mm_body,
        grid=(_GRID,),
        in_specs=[
            pl.BlockSpec((_BLK, IN_FEATS), lambda i: (i, 0)),
            pl.BlockSpec((IN_FEATS, NUM_CLASSES), lambda i: (0, 0)),
        ],
        out_specs=pl.BlockSpec((_BLK, LANES), lambda i: (i, 0)),
        out_shape=jax.ShapeDtypeStruct((N_PAD, LANES), jnp.float32),
    )(in_feat, W)

    y = _scale(xw, ds0, ds1)
    a0, a1, d0, d1 = _agg(y, edge3)
    return _comb(a0, a1, d0, d1, b)[:N_NODES]


# speedup vs baseline: 46.6597x; 1.0144x over previous
import functools

import jax
import jax.numpy as jnp
from jax import lax
from jax.experimental import pallas as pl
from jax.experimental.pallas import tpu as pltpu
from jax.experimental.pallas import tpu_sc as plsc

N_NODES = 10000
N_EDGES = 320000
IN_FEATS = 128
NUM_CLASSES = 16

N_PAD = 10240
NW = 32
LANES = 128
E_ROWS = N_EDGES // LANES
BASE_STEPS = E_ROWS // NW
TAIL_ROWS = E_ROWS - BASE_STEPS * NW
ROWS_PER_TILE = N_PAD // 16
NODES_PER_TILE = N_PAD // NW
NBUF = 16

_mesh = plsc.VectorSubcoreMesh(core_axis_name="c", subcore_axis_name="s")
_sc_params = pltpu.CompilerParams(use_tc_tiling_on_sc=False)

_PART = jax.ShapeDtypeStruct((N_PAD, NUM_CLASSES), jnp.float32)


def _wid(cid, sid):
    return cid * 16 + sid


def _load_idx(edge_hbm, which, gid, idx):
    pltpu.sync_copy(edge_hbm.at[which, pl.ds(gid * BASE_STEPS, BASE_STEPS)],
                    idx.at[pl.ds(0, BASE_STEPS)])

    @pl.when(gid < TAIL_ROWS)
    def _():
        pltpu.sync_copy(
            edge_hbm.at[which, pl.ds(NW * BASE_STEPS + gid, 1)],
            idx.at[pl.ds(BASE_STEPS, 1)])


def _rsqrt_norm(d16):
    x = jnp.maximum(d16, 1.0)
    t = x
    r = jnp.full((16,), 1.0, jnp.float32)
    for j in range(9, 0, -1):
        c = t >= float(4 ** j)
        t = jnp.where(c, t * float(4.0 ** (-j)), t)
        r = jnp.where(c, r * float(2.0 ** (-j)), r)
    c = t >= 2.0
    t = jnp.where(c, t * 0.5, t)
    r = jnp.where(c, r * 0.7071067811865476, r)
    s = jnp.full((16,), 0.85, jnp.float32)
    for _ in range(4):
        s = s * (1.5 - 0.5 * t * s * s)
    return jnp.where(d16 > 0, s * r, 0.0)


@functools.partial(
    pl.kernel,
    mesh=_mesh,
    out_type=[_PART, _PART],
    scratch_types=[
        pltpu.VMEM((BASE_STEPS + 1, LANES), jnp.int32),
        pltpu.VMEM((LANES, NUM_CLASSES), jnp.float32),
        pltpu.VMEM((ROWS_PER_TILE, NUM_CLASSES), jnp.float32),
        pltpu.VMEM_SHARED((N_PAD, NUM_CLASSES), jnp.float32),
    ],
    compiler_params=_sc_params,
)
def _deg(edge_hbm, out_s0, out_s1, idx_s, ones_v, zeros_v, hist_s):
    cid = lax.axis_index("c")
    sid = lax.axis_index("s")
    gid = _wid(cid, sid)
    nsteps = BASE_STEPS + jnp.where(gid < TAIL_ROWS, 1, 0)

    def fill(i, _):
        ones_v[i] = jnp.ones((16,), jnp.float32)
        zeros_v[i] = jnp.zeros((16,), jnp.float32)
        return 0
    lax.fori_loop(0, LANES, fill, 0)

    def fillz(i, _):
        zeros_v[i] = jnp.zeros((16,), jnp.float32)
        return 0
    lax.fori_loop(LANES, ROWS_PER_TILE, fillz, 0)

    base = pl.multiple_of(sid * ROWS_PER_TILE, 128)
    pltpu.sync_copy(zeros_v, hist_s.at[pl.ds(base, ROWS_PER_TILE)])
    _load_idx(edge_hbm, 0, gid, idx_s)
    plsc.subcore_barrier()

    def body(s, _):
        pltpu.sync_copy(ones_v, hist_s.at[idx_s.at[s]], add=True)
        return 0
    lax.fori_loop(0, nsteps, body, 0)

    plsc.subcore_barrier()

    @pl.when(cid == 0)
    def _():
        pltpu.sync_copy(hist_s.at[pl.ds(base, ROWS_PER_TILE)],
                        out_s0.at[pl.ds(base, ROWS_PER_TILE)])

    @pl.when(cid == 1)
    def _():
        pltpu.sync_copy(hist_s.at[pl.ds(base, ROWS_PER_TILE)],
                        out_s1.at[pl.ds(base, ROWS_PER_TILE)])


def _mm_body(x_ref, w_ref, y_ref):
    y_ref[...] = jnp.dot(x_ref[...], w_ref[...],
                         preferred_element_type=jnp.float32)


@functools.partial(
    pl.kernel,
    mesh=_mesh,
    out_type=_PART,
    scratch_types=[
        pltpu.VMEM((NODES_PER_TILE, NUM_CLASSES), jnp.float32),
        pltpu.VMEM((NODES_PER_TILE, NUM_CLASSES), jnp.float32),
        pltpu.VMEM((NODES_PER_TILE, NUM_CLASSES), jnp.float32),
        pltpu.VMEM((NODES_PER_TILE, NUM_CLASSES), jnp.float32),
    ],
    compiler_params=_sc_params,
)
def _scale(xw_hbm, ds0_hbm, ds1_hbm, y_hbm, xv, d0v, d1v, yv):
    cid = lax.axis_index("c")
    sid = lax.axis_index("s")
    gid = _wid(cid, sid)
    start = pl.multiple_of(gid * NODES_PER_TILE, 128)
    pltpu.sync_copy(xw_hbm.at[pl.ds(start, NODES_PER_TILE)], xv)
    pltpu.sync_copy(ds0_hbm.at[pl.ds(start, NODES_PER_TILE)], d0v)
    pltpu.sync_copy(ds1_hbm.at[pl.ds(start, NODES_PER_TILE)], d1v)

    def body(i, _):
        norm = _rsqrt_norm(d0v[i] + d1v[i])
        yv[i] = xv[i] * norm
        return 0
    lax.fori_loop(0, NODES_PER_TILE, body, 0)

    pltpu.sync_copy(yv, y_hbm.at[pl.ds(start, NODES_PER_TILE)])


@functools.partial(
    pl.kernel,
    mesh=_mesh,
    out_type=[_PART, _PART, _PART, _PART],
    scratch_types=[
        pltpu.VMEM((BASE_STEPS + 1, LANES), jnp.int32),
        pltpu.VMEM((BASE_STEPS + 1, LANES), jnp.int32),
        pltpu.VMEM((NBUF, LANES, NUM_CLASSES), jnp.float32),
        pltpu.VMEM((LANES, NUM_CLASSES), jnp.float32),
        pltpu.VMEM((ROWS_PER_TILE, NUM_CLASSES), jnp.float32),
        pltpu.VMEM_SHARED((N_PAD, NUM_CLASSES), jnp.float32),
        pltpu.VMEM_SHARED((N_PAD, NUM_CLASSES), jnp.float32),
        pltpu.SemaphoreType.DMA,
        pltpu.SemaphoreType.DMA,
    ],
    compiler_params=_sc_params,
)
def _agg(y_hbm, edge_hbm, out_a0, out_a1, out_d0, out_d1, idx_s, idx_d, rows,
         ones_v, zeros_v, agg, hist_d, sem, dsem):
    cid = lax.axis_index("c")
    sid = lax.axis_index("s")
    gid = _wid(cid, sid)
    nsteps = BASE_STEPS + jnp.where(gid < TAIL_ROWS, 1, 0)

    def fill(i, _):
        ones_v[i] = jnp.ones((16,), jnp.float32)
        zeros_v[i] = jnp.zeros((16,), jnp.float32)
        return 0
    lax.fori_loop(0, LANES, fill, 0)

    def fillz(i, _):
        zeros_v[i] = jnp.zeros((16,), jnp.float32)
        return 0
    lax.fori_loop(LANES, ROWS_PER_TILE, fillz, 0)

    base = pl.multiple_of(sid * ROWS_PER_TILE, 128)
    pltpu.sync_copy(zeros_v, agg.at[pl.ds(base, ROWS_PER_TILE)])
    pltpu.sync_copy(zeros_v, hist_d.at[pl.ds(base, ROWS_PER_TILE)])
    _load_idx(edge_hbm, 0, gid, idx_s)
    _load_idx(edge_hbm, 1, gid, idx_d)
    plsc.subcore_barrier()

    for d in range(NBUF):
        pltpu.async_copy(y_hbm.at[idx_s.at[d]], rows.at[d], sem)

    def body(s, _):
        buf = lax.rem(s, NBUF)
        pltpu.make_async_copy(y_hbm.at[idx_s.at[s]], rows.at[buf], sem).wait()
        d1 = pltpu.async_copy(ones_v, hist_d.at[idx_d.at[s]], dsem, add=True)
        pltpu.sync_copy(rows.at[buf], agg.at[idx_d.at[s]], add=True)

        @pl.when(s + NBUF < nsteps)
        def _():
            pltpu.async_copy(y_hbm.at[idx_s.at[s + NBUF]], rows.at[buf], sem)
        d1.wait()
        return 0
    lax.fori_loop(0, nsteps, body, 0)

    plsc.subcore_barrier()

    @pl.when(cid == 0)
    def _():
        pltpu.sync_copy(agg.at[pl.ds(base, ROWS_PER_TILE)],
                        out_a0.at[pl.ds(base, ROWS_PER_TILE)])
        pltpu.sync_copy(hist_d.at[pl.ds(base, ROWS_PER_TILE)],
                        out_d0.at[pl.ds(base, ROWS_PER_TILE)])

    @pl.when(cid == 1)
    def _():
        pltpu.sync_copy(agg.at[pl.ds(base, ROWS_PER_TILE)],
                        out_a1.at[pl.ds(base, ROWS_PER_TILE)])
        pltpu.sync_copy(hist_d.at[pl.ds(base, ROWS_PER_TILE)],
                        out_d1.at[pl.ds(base, ROWS_PER_TILE)])


@functools.partial(
    pl.kernel,
    mesh=_mesh,
    out_type=jax.ShapeDtypeStruct((N_PAD, NUM_CLASSES), jnp.float32),
    scratch_types=[
        pltpu.VMEM((NODES_PER_TILE, NUM_CLASSES), jnp.float32),
        pltpu.VMEM((NODES_PER_TILE, NUM_CLASSES), jnp.float32),
        pltpu.VMEM((NODES_PER_TILE, NUM_CLASSES), jnp.float32),
        pltpu.VMEM((NODES_PER_TILE, NUM_CLASSES), jnp.float32),
        pltpu.VMEM((NODES_PER_TILE, NUM_CLASSES), jnp.float32),
        pltpu.VMEM((NUM_CLASSES,), jnp.float32),
    ],
    compiler_params=_sc_params,
)
def _comb(a0_hbm, a1_hbm, d0_hbm, d1_hbm, b_hbm, h_hbm,
          a0v, a1v, d0v, d1v, hv, bv):
    cid = lax.axis_index("c")
    sid = lax.axis_index("s")
    gid = _wid(cid, sid)
    start = pl.multiple_of(gid * NODES_PER_TILE, 128)
    pltpu.sync_copy(a0_hbm.at[pl.ds(start, NODES_PER_TILE)], a0v)
    pltpu.sync_copy(a1_hbm.at[pl.ds(start, NODES_PER_TILE)], a1v)
    pltpu.sync_copy(d0_hbm.at[pl.ds(start, NODES_PER_TILE)], d0v)
    pltpu.sync_copy(d1_hbm.at[pl.ds(start, NODES_PER_TILE)], d1v)
    pltpu.sync_copy(b_hbm, bv)
    b16 = bv[...]

    def body(i, _):
        norm = _rsqrt_norm(d0v[i] + d1v[i])
        hv[i] = jnp.maximum((a0v[i] + a1v[i]) * norm + b16, 0.0)
        return 0
    lax.fori_loop(0, NODES_PER_TILE, body, 0)

    pltpu.sync_copy(hv, h_hbm.at[pl.ds(start, NODES_PER_TILE)])


_BLK = 1280
_GRID = N_PAD // _BLK


def kernel(in_feat, edge_index, W, b):
    edge3 = edge_index.reshape(2, E_ROWS, LANES)

    ds0, ds1 = _deg(edge3)

    xw = pl.pallas_call(
        _mm_body,
        grid=(_GRID,),
        in_specs=[
            pl.BlockSpec((_BLK, IN_FEATS), lambda i: (i, 0)),
            pl.BlockSpec((IN_FEATS, NUM_CLASSES), lambda i: (0, 0)),
        ],
        out_specs=pl.BlockSpec((_BLK, NUM_CLASSES), lambda i: (i, 0)),
        out_shape=jax.ShapeDtypeStruct((N_PAD, NUM_CLASSES), jnp.float32),
    )(in_feat, W)

    y = _scale(xw, ds0, ds1)
    a0, a1, d0, d1 = _agg(y, edge3)
    return _comb(a0, a1, d0, d1, b)[:N_NODES]
